# baseline (speedup 1.0000x reference)
import jax
import jax.numpy as jnp
from jax.experimental import pallas as pl
from jax.experimental.pallas import tpu as pltpu

_FILTERS = 64
_C = 64
_D = _H = _W = 32
_N = _D * _H * _W
_ZPAD = 1088
_NROW = _N + 2 * _ZPAD
_EXT = 16
_TB = 2048
_G = 16
_CH = 4368

_GROUPS = tuple((dz, dy) for dz in (-1, 0, 1) for dy in (-1, 0, 1))


def _body(feat_hbm, w_ref, b_ref, idx_ref, out_ref, fz_ref, a_ref, bb_ref, sem):
    g = pl.program_id(0)

    @pl.when(g == 0)
    def _build():
        fz_ref[pl.ds(0, _EXT + _ZPAD), :] = jnp.zeros((_EXT + _ZPAD, _C), jnp.float32)
        fz_ref[pl.ds(_EXT + _ZPAD + _N, _ZPAD + _EXT), :] = jnp.zeros(
            (_ZPAD + _EXT, _C), jnp.float32)
        cp = pltpu.make_async_copy(
            feat_hbm, fz_ref.at[pl.ds(_EXT + _ZPAD, _N), :], sem)
        cp.start()
        cp.wait()
        for c in range(8):
            r0 = c * _CH
            xr = jnp.bitwise_and(
                jax.lax.broadcasted_iota(jnp.int32, (_CH, 1), 0) + r0, 31)
            fm = fz_ref[pl.ds(_EXT + r0 - 1, _CH), :]
            f0 = fz_ref[pl.ds(_EXT + r0, _CH), :]
            fp = fz_ref[pl.ds(_EXT + r0 + 1, _CH), :]
            a_ref[pl.ds(r0, _CH), :] = jnp.concatenate(
                [(fm * (xr >= 1)).astype(jnp.bfloat16),
                 f0.astype(jnp.bfloat16)], axis=1)
            bb_ref[pl.ds(r0, _CH), :] = (fp * (xr <= 30)).astype(jnp.bfloat16)

    base = _ZPAD + g * _TB
    i = jax.lax.broadcasted_iota(jnp.int32, (_TB, 1), 0) + g * _TB
    y = jnp.bitwise_and(jax.lax.shift_right_logical(i, 5), 31)
    my = {-1: (y >= 1).astype(jnp.bfloat16), 1: (y <= 30).astype(jnp.bfloat16)}
    acc = None
    for dz, dy in _GROUPS:
        off = dz * 1024 + dy * 32
        bk = ((dz + 1) * 9 + (dy + 1) * 3) * _C
        a_op = a_ref[pl.ds(base + off, _TB), :]
        b_op = bb_ref[pl.ds(base + off, _TB), :]
        if dy != 0:
            a_op = a_op * my[dy]
            b_op = b_op * my[dy]
        part = jnp.dot(a_op, w_ref[pl.ds(bk, 2 * _C), :],
                       preferred_element_type=jnp.float32)
        part += jnp.dot(b_op, w_ref[pl.ds(bk + 2 * _C, _C), :],
                        preferred_element_type=jnp.float32)
        acc = part if acc is None else acc + part
    act = (idx_ref[...] != 0).astype(jnp.float32)
    out_ref[...] = (acc + b_ref[...]) * act


def kernel(feat, index, w, b):
    out = pl.pallas_call(
        _body,
        grid=(_G,),
        in_specs=[
            pl.BlockSpec(memory_space=pltpu.MemorySpace.HBM),
            pl.BlockSpec((27 * _C, _FILTERS), lambda g: (0, 0)),
            pl.BlockSpec((1, _FILTERS), lambda g: (0, 0)),
            pl.BlockSpec((_TB, 1), lambda g: (g, 0)),
        ],
        out_specs=pl.BlockSpec((_TB, _FILTERS), lambda g: (g, 0)),
        out_shape=jax.ShapeDtypeStruct((_N, _FILTERS), jnp.float32),
        scratch_shapes=[
            pltpu.VMEM((_EXT + _NROW + _EXT, _C), jnp.float32),
            pltpu.VMEM((_NROW, 2 * _C), jnp.bfloat16),
            pltpu.VMEM((_NROW, _C), jnp.bfloat16),
            pltpu.SemaphoreType.DMA,
        ],
    )(feat.reshape(_N, _C), w.astype(jnp.bfloat16),
      b.reshape(1, _FILTERS), index.reshape(_N, 1))
    return out.reshape(1, _D, _H, _W, _FILTERS)


# speedup vs baseline: 1.0285x; 1.0285x over previous
import jax
import jax.numpy as jnp
from jax.experimental import pallas as pl
from jax.experimental.pallas import tpu as pltpu

_FILTERS = 64
_C = 64
_D = _H = _W = 32
_N = _D * _H * _W
_ZPAD = 1088
_NROW = _N + 2 * _ZPAD
_EXT = 16
_TB = 2048
_G = 16
_CH = 4368

_GROUPS = tuple((dz, dy) for dz in (-1, 0, 1) for dy in (-1, 0, 1))


def _body(feat_hbm, wa_ref, wb_ref, b_ref, idx_ref, out_ref,
          fz_ref, a_ref, bb_ref, sem):
    g = pl.program_id(0)

    @pl.when(g == 0)
    def _build():
        fz_ref[pl.ds(0, _EXT + _ZPAD), :] = jnp.zeros((_EXT + _ZPAD, _C), jnp.float32)
        fz_ref[pl.ds(_EXT + _ZPAD + _N, _ZPAD + _EXT), :] = jnp.zeros(
            (_ZPAD + _EXT, _C), jnp.float32)
        cp = pltpu.make_async_copy(
            feat_hbm, fz_ref.at[pl.ds(_EXT + _ZPAD, _N), :], sem)
        cp.start()
        cp.wait()
        for c in range(8):
            r0 = c * _CH
            xr = jnp.bitwise_and(
                jax.lax.broadcasted_iota(jnp.int32, (_CH, 1), 0) + r0, 31)
            fm = fz_ref[pl.ds(_EXT + r0 - 1, _CH), :]
            f0 = fz_ref[pl.ds(_EXT + r0, _CH), :]
            fp = fz_ref[pl.ds(_EXT + r0 + 1, _CH), :]
            a_ref[pl.ds(r0, _CH), :] = jnp.concatenate(
                [(fm * (xr >= 1)).astype(jnp.bfloat16),
                 f0.astype(jnp.bfloat16)], axis=1)
            bb_ref[pl.ds(r0, _CH), :] = (fp * (xr <= 30)).astype(jnp.bfloat16)

    base = _ZPAD + g * _TB
    i = jax.lax.broadcasted_iota(jnp.int32, (_TB, 1), 0) + g * _TB
    y = jnp.bitwise_and(jax.lax.shift_right_logical(i, 5), 31)
    acc = None
    for dy in (-1, 0, 1):
        part = None
        for dz in (-1, 0, 1):
            j = (dz + 1) * 3 + (dy + 1)
            off = dz * 1024 + dy * 32
            p = jnp.dot(a_ref[pl.ds(base + off, _TB), :], wa_ref[j],
                        preferred_element_type=jnp.float32)
            p += jnp.dot(bb_ref[pl.ds(base + off, _TB), :], wb_ref[j],
                         preferred_element_type=jnp.float32)
            part = p if part is None else part + p
        if dy != 0:
            my = (y >= 1) if dy < 0 else (y <= 30)
            part *= my.astype(jnp.float32)
        acc = part if acc is None else acc + part
    act = (idx_ref[...] != 0).astype(jnp.float32)
    out_ref[...] = (acc + b_ref[...]) * act


def kernel(feat, index, w, b):
    w5 = w.reshape(3, 3, 3, _C, _FILTERS).astype(jnp.bfloat16)
    wa = w5[:, :, 0:2].reshape(9, 2 * _C, _FILTERS)
    wb = w5[:, :, 2].reshape(9, _C, _FILTERS)

    out = pl.pallas_call(
        _body,
        grid=(_G,),
        in_specs=[
            pl.BlockSpec(memory_space=pltpu.MemorySpace.HBM),
            pl.BlockSpec((9, 2 * _C, _FILTERS), lambda g: (0, 0, 0)),
            pl.BlockSpec((9, _C, _FILTERS), lambda g: (0, 0, 0)),
            pl.BlockSpec((1, _FILTERS), lambda g: (0, 0)),
            pl.BlockSpec((_TB, 1), lambda g: (g, 0)),
        ],
        out_specs=pl.BlockSpec((_TB, _FILTERS), lambda g: (g, 0)),
        out_shape=jax.ShapeDtypeStruct((_N, _FILTERS), jnp.float32),
        scratch_shapes=[
            pltpu.VMEM((_EXT + _NROW + _EXT, _C), jnp.float32),
            pltpu.VMEM((_NROW, 2 * _C), jnp.bfloat16),
            pltpu.VMEM((_NROW, _C), jnp.bfloat16),
            pltpu.SemaphoreType.DMA,
        ],
    )(feat.reshape(_N, _C), wa, wb, b.reshape(1, _FILTERS),
      index.reshape(_N, 1))
    return out.reshape(1, _D, _H, _W, _FILTERS)
